# Initial kernel scaffold; baseline (speedup 1.0000x reference)
import jax
import jax.numpy as jnp
from jax.experimental import pallas as pl


def kernel(x, edge_index, edge_attr, batch, params):
    raise NotImplementedError("write your pallas kernel here")


# speedup vs baseline: 1.1293x; 1.1293x over previous
import functools

import jax
import jax.numpy as jnp
from jax import lax
from jax.experimental import pallas as pl
from jax.experimental.pallas import tpu as pltpu
from jax.experimental.pallas import tpu_sc as plsc

N = 10000
E = 160000
B = 64
NF = 128
P1 = 128
P2 = 32
NL = 3

NW = 32
CH = 128
EPW = 5120
E_PAD = NW * EPW
NCHUNK = EPW // CH
N_ACC = 10240
EB = 2048
NEB = E_PAD // EB
NB_N = 10
NBR = N // NB_N


def _leaky(t):
    return jnp.where(t >= 0, t, 0.01 * t)


def _pre1_body(x_ref, w_ref, b_ref, u_ref, s_ref, q_ref):
    i = pl.program_id(0)
    u = jnp.dot(x_ref[...], w_ref[...], preferred_element_type=jnp.float32)
    u = u + b_ref[...]
    u_ref[...] = u

    @pl.when(i == 0)
    def _():
        s_ref[...] = jnp.zeros_like(s_ref)
        q_ref[...] = jnp.zeros_like(q_ref)

    s_ref[...] += jnp.sum(u, axis=0, keepdims=True)
    q_ref[...] += jnp.sum(u * u, axis=0, keepdims=True)


def _pre1(x, w1t, b1):
    return pl.pallas_call(
        _pre1_body,
        grid=(NB_N,),
        in_specs=[
            pl.BlockSpec((NBR, NF), lambda i: (i, 0)),
            pl.BlockSpec((NF, 80), lambda i: (0, 0)),
            pl.BlockSpec((1, 80), lambda i: (0, 0)),
        ],
        out_specs=[
            pl.BlockSpec((NBR, 80), lambda i: (i, 0)),
            pl.BlockSpec((1, 80), lambda i: (0, 0)),
            pl.BlockSpec((1, 80), lambda i: (0, 0)),
        ],
        out_shape=[
            jax.ShapeDtypeStruct((N, 80), jnp.float32),
            jax.ShapeDtypeStruct((1, 80), jnp.float32),
            jax.ShapeDtypeStruct((1, 80), jnp.float32),
        ],
    )(x, w1t, b1)


def _pre2_body(u_ref, ac_ref, w_ref, b_ref, u2_ref, s_ref, q_ref):
    i = pl.program_id(0)
    h0 = _leaky(u_ref[...] * ac_ref[0:1, :] + ac_ref[1:2, :])
    u2 = jnp.dot(h0, w_ref[...], preferred_element_type=jnp.float32) + b_ref[...]
    u2_ref[...] = u2

    @pl.when(i == 0)
    def _():
        s_ref[...] = jnp.zeros_like(s_ref)
        q_ref[...] = jnp.zeros_like(q_ref)

    s_ref[...] += jnp.sum(u2, axis=0, keepdims=True)
    q_ref[...] += jnp.sum(u2 * u2, axis=0, keepdims=True)


def _pre2(u1, ac, w2t, b2):
    return pl.pallas_call(
        _pre2_body,
        grid=(NB_N,),
        in_specs=[
            pl.BlockSpec((NBR, 80), lambda i: (i, 0)),
            pl.BlockSpec((2, 80), lambda i: (0, 0)),
            pl.BlockSpec((80, P2), lambda i: (0, 0)),
            pl.BlockSpec((1, P2), lambda i: (0, 0)),
        ],
        out_specs=[
            pl.BlockSpec((NBR, P2), lambda i: (i, 0)),
            pl.BlockSpec((1, P2), lambda i: (0, 0)),
            pl.BlockSpec((1, P2), lambda i: (0, 0)),
        ],
        out_shape=[
            jax.ShapeDtypeStruct((N, P2), jnp.float32),
            jax.ShapeDtypeStruct((1, P2), jnp.float32),
            jax.ShapeDtypeStruct((1, P2), jnp.float32),
        ],
    )(u1, ac, w2t, b2)


def _prefin_body(u_ref, ac_ref, o_ref):
    o_ref[...] = _leaky(u_ref[...] * ac_ref[0:1, :] + ac_ref[1:2, :])


def _prefin(u2, ac):
    return pl.pallas_call(
        _prefin_body,
        out_shape=jax.ShapeDtypeStruct((N, P2), jnp.float32),
    )(u2, ac)


def _e1stats_body(r_ref, w01_ref, s_ref, q_ref):
    i = pl.program_id(0)
    r = r_ref[...]
    pre = r * w01_ref[0:1, :] + (1.0 - r) * w01_ref[1:2, :]
    row = i * EB + lax.broadcasted_iota(jnp.int32, (EB, 1), 0)
    pre = pre * (row < E).astype(jnp.float32)

    @pl.when(i == 0)
    def _():
        s_ref[...] = jnp.zeros_like(s_ref)
        q_ref[...] = jnp.zeros_like(q_ref)

    s_ref[...] += jnp.sum(pre, axis=0, keepdims=True)
    q_ref[...] += jnp.sum(pre * pre, axis=0, keepdims=True)


def _e1stats(ea_p, w01):
    return pl.pallas_call(
        _e1stats_body,
        grid=(NEB,),
        in_specs=[
            pl.BlockSpec((EB, 1), lambda i: (i, 0)),
            pl.BlockSpec((2, P1), lambda i: (0, 0)),
        ],
        out_specs=[
            pl.BlockSpec((1, P1), lambda i: (0, 0)),
            pl.BlockSpec((1, P1), lambda i: (0, 0)),
        ],
        out_shape=[
            jax.ShapeDtypeStruct((1, P1), jnp.float32),
            jax.ShapeDtypeStruct((1, P1), jnp.float32),
        ],
    )(ea_p, w01)


def _e1mat_body(r_ref, w01_ref, ac_ref, e1_ref, g_ref, m_ref):
    i = pl.program_id(0)
    r = r_ref[...]
    pre = r * w01_ref[0:1, :] + (1.0 - r) * w01_ref[1:2, :]
    e1 = _leaky(pre * ac_ref[0:1, :] + ac_ref[1:2, :])
    row = i * EB + lax.broadcasted_iota(jnp.int32, (EB, 1), 0)
    e1 = e1 * (row < E).astype(jnp.float32)
    e1_ref[...] = e1

    @pl.when(i == 0)
    def _():
        g_ref[...] = jnp.zeros_like(g_ref)
        m_ref[...] = jnp.zeros_like(m_ref)

    g_ref[...] += lax.dot_general(e1, e1, (((0,), (0,)), ((), ())),
                                  preferred_element_type=jnp.float32)
    m_ref[...] += jnp.sum(e1, axis=0, keepdims=True)


def _e1mat(ea_p, w01, ac):
    return pl.pallas_call(
        _e1mat_body,
        grid=(NEB,),
        in_specs=[
            pl.BlockSpec((EB, 1), lambda i: (i, 0)),
            pl.BlockSpec((2, P1), lambda i: (0, 0)),
            pl.BlockSpec((2, P1), lambda i: (0, 0)),
        ],
        out_specs=[
            pl.BlockSpec((EB, P1), lambda i: (i, 0)),
            pl.BlockSpec((P1, P1), lambda i: (0, 0)),
            pl.BlockSpec((1, P1), lambda i: (0, 0)),
        ],
        out_shape=[
            jax.ShapeDtypeStruct((E_PAD, P1), jnp.float32),
            jax.ShapeDtypeStruct((P1, P1), jnp.float32),
            jax.ShapeDtypeStruct((1, P1), jnp.float32),
        ],
    )(ea_p, w01, ac)


def _e2aff_body(g_ref, m_ref, w2_ref, g2_ref, b2_ref, a_ref, c_ref):
    w2 = w2_ref[...]
    mu = lax.dot_general(w2, m_ref[...], (((1,), (1,)), ((), ())),
                         preferred_element_type=jnp.float32) / E
    t = lax.dot_general(w2, g_ref[...], (((1,), (0,)), ((), ())),
                        preferred_element_type=jnp.float32)
    ssq = jnp.sum(t * w2, axis=1, keepdims=True) / E
    var = ssq - mu * mu
    a = g2_ref[...] / jnp.sqrt(var + 1e-5)
    a_ref[...] = a
    c_ref[...] = b2_ref[...] - mu * a


def _e2aff(g, m, w2, g2c, b2c):
    return pl.pallas_call(
        _e2aff_body,
        out_shape=[
            jax.ShapeDtypeStruct((P2 * P2, 1), jnp.float32),
            jax.ShapeDtypeStruct((P2 * P2, 1), jnp.float32),
        ],
    )(g, m, w2, g2c, b2c)


def _msg_body(e1_ref, hs_ref, w2a_ref, cmat_ref, msg_ref):
    e2a = jnp.dot(e1_ref[...], w2a_ref[...],
                  preferred_element_type=jnp.float32)
    hs = hs_ref[...]
    acc = jnp.dot(hs, cmat_ref[...], preferred_element_type=jnp.float32)
    for i in range(P2):
        acc += hs[:, i:i + 1] * e2a[:, i * P2:(i + 1) * P2]
    msg_ref[...] = acc


def _msg(e1_bf, hs, w2a, cmat):
    return pl.pallas_call(
        _msg_body,
        grid=(NEB,),
        in_specs=[
            pl.BlockSpec((EB, P1), lambda i: (i, 0)),
            pl.BlockSpec((EB, P2), lambda i: (i, 0)),
            pl.BlockSpec((P1, P2 * P2), lambda i: (0, 0)),
            pl.BlockSpec((P2, P2), lambda i: (0, 0)),
        ],
        out_specs=pl.BlockSpec((EB, P2), lambda i: (i, 0)),
        out_shape=jax.ShapeDtypeStruct((E_PAD, P2), jnp.float32),
    )(e1_bf, hs, w2a, cmat)


def _gru_body(aggp_ref, degp_ref, h_ref, crw_ref, cb_ref, wih_ref, whh_ref,
              bih_ref, bhh_ref, out_ref):
    agg = aggp_ref[0, 0:N, :] + aggp_ref[1, 0:N, :]
    deg = degp_ref[0, 0:N, 0:1] + degp_ref[1, 0:N, 0:1]
    invd = 1.0 / jnp.maximum(deg, 1.0)
    h = h_ref[...]
    m = _leaky(agg * invd
               + jnp.dot(h, crw_ref[...], preferred_element_type=jnp.float32)
               + cb_ref[...])
    gi = jnp.dot(m, wih_ref[...], preferred_element_type=jnp.float32) + bih_ref[...]
    gh = jnp.dot(h, whh_ref[...], preferred_element_type=jnp.float32) + bhh_ref[...]
    r = jax.nn.sigmoid(gi[:, :P2] + gh[:, :P2])
    z = jax.nn.sigmoid(gi[:, P2:2 * P2] + gh[:, P2:2 * P2])
    n = jnp.tanh(gi[:, 2 * P2:] + r * gh[:, 2 * P2:])
    out_ref[...] = (1.0 - z) * n + z * h


def _gru(aggp, degp, h, crw, cb, wiht, whht, bih, bhh):
    return pl.pallas_call(
        _gru_body,
        out_shape=jax.ShapeDtypeStruct((N, P2), jnp.float32),
    )(aggp, degp, h, crw, cb, wiht, whht, bih, bhh)


def _s2s_body(h_ref, brow_ref, bcol_ref, wih_ref, whh_ref, bih_ref, bhh_ref,
              l1_ref, l1b_ref, l2_ref, l2b_ref, lf_ref, lfb_ref, y_ref):
    h = h_ref[...]
    iota_bn = lax.broadcasted_iota(jnp.int32, (B, N), 0).astype(jnp.float32)
    oh_t = (brow_ref[...] == iota_bn).astype(jnp.float32)
    iota_nb = lax.broadcasted_iota(jnp.int32, (N, B), 1).astype(jnp.float32)
    oh = (bcol_ref[...] == iota_nb).astype(jnp.float32)
    q_star = jnp.zeros((B, 2 * P2), jnp.float32)
    hl = jnp.zeros((B, P2), jnp.float32)
    cl = jnp.zeros((B, P2), jnp.float32)
    for _ in range(3):
        g = (jnp.dot(q_star, wih_ref[...], preferred_element_type=jnp.float32)
             + bih_ref[...]
             + jnp.dot(hl, whh_ref[...], preferred_element_type=jnp.float32)
             + bhh_ref[...])
        i_g = jax.nn.sigmoid(g[:, :P2])
        f_g = jax.nn.sigmoid(g[:, P2:2 * P2])
        g_g = jnp.tanh(g[:, 2 * P2:3 * P2])
        o_g = jax.nn.sigmoid(g[:, 3 * P2:])
        cl = f_g * cl + i_g * g_g
        hl = o_g * jnp.tanh(cl)
        q = hl
        qb = jnp.dot(oh, q, preferred_element_type=jnp.float32)
        e = jnp.sum(h * qb, axis=1, keepdims=True)
        ex = jnp.exp(e - jnp.max(e))
        zc = jnp.concatenate([ex, ex * h], axis=1)
        s = jnp.dot(oh_t, zc, preferred_element_type=jnp.float32)
        r_ = s[:, 1:] / jnp.maximum(s[:, 0:1], 1e-30)
        q_star = jnp.concatenate([q, r_], axis=1)
    y = _leaky(jnp.dot(q_star, l1_ref[...], preferred_element_type=jnp.float32)
               + l1b_ref[...])
    y = _leaky(jnp.dot(y, l2_ref[...], preferred_element_type=jnp.float32)
               + l2b_ref[...])
    y_ref[...] = (jnp.dot(y, lf_ref[...], preferred_element_type=jnp.float32)
                  + lfb_ref[...])


def _s2s(h, brow, bcol, wiht, whht, bih, bhh, l1t, l1b, l2t, l2b, lft, lfb):
    return pl.pallas_call(
        _s2s_body,
        out_shape=jax.ShapeDtypeStruct((B, 1), jnp.float32),
    )(h, brow, bcol, wiht, whht, bih, bhh, l1t, l1b, l2t, l2b, lft, lfb)


def _sc_mesh():
    return plsc.VectorSubcoreMesh(core_axis_name="c", subcore_axis_name="s",
                                  num_cores=2, num_subcores=16)


@functools.lru_cache(maxsize=None)
def _sc_gather_kernel():
    @functools.partial(
        pl.kernel,
        out_type=jax.ShapeDtypeStruct((E_PAD, P2), jnp.float32),
        mesh=_sc_mesh(),
        compiler_params=pltpu.CompilerParams(use_tc_tiling_on_sc=False),
        scratch_types=[
            pltpu.VMEM((CH,), jnp.int32),
            pltpu.VMEM((CH, P2), jnp.float32),
            pltpu.SemaphoreType.DMA,
        ],
    )
    def _k(h_hbm, idx_hbm, out_hbm, idx_v, rows_v, sem):
        wid = lax.axis_index("s") * 2 + lax.axis_index("c")

        def chunk(j, carry):
            base = wid * EPW + j * CH
            pltpu.sync_copy(idx_hbm.at[pl.ds(base, CH)], idx_v)
            pltpu.async_copy(h_hbm.at[idx_v], rows_v, sem).wait()
            pltpu.sync_copy(rows_v, out_hbm.at[pl.ds(base, CH)])
            return carry

        lax.fori_loop(0, NCHUNK, chunk, 0)

    return _k


def _sc_gather(h, idx):
    return _sc_gather_kernel()(h, idx)


@functools.lru_cache(maxsize=None)
def _sc_scatter_kernel():
    @functools.partial(
        pl.kernel,
        out_type=jax.ShapeDtypeStruct((2, N_ACC, P2), jnp.float32),
        mesh=_sc_mesh(),
        compiler_params=pltpu.CompilerParams(use_tc_tiling_on_sc=False),
        scratch_types=[
            pltpu.VMEM((CH,), jnp.int32),
            pltpu.VMEM((CH, P2), jnp.float32),
            pltpu.VMEM_SHARED((N_ACC, P2), jnp.float32),
        ],
    )
    def _k(msg_hbm, idx_hbm, zero_hbm, out_hbm, idx_v, rows_v, acc):
        cid = lax.axis_index("c")
        sid = lax.axis_index("s")
        wid = sid * 2 + cid
        rows_per = N_ACC // 16
        pltpu.sync_copy(zero_hbm.at[pl.ds(sid * rows_per, rows_per)],
                        acc.at[pl.ds(sid * rows_per, rows_per)])
        plsc.subcore_barrier()

        def chunk(j, carry):
            base = wid * EPW + j * CH
            pltpu.sync_copy(idx_hbm.at[pl.ds(base, CH)], idx_v)
            pltpu.sync_copy(msg_hbm.at[pl.ds(base, CH)], rows_v)
            pltpu.sync_copy(rows_v, acc.at[idx_v], add=True)
            return carry

        lax.fori_loop(0, NCHUNK, chunk, 0)
        plsc.subcore_barrier()
        pltpu.sync_copy(acc.at[pl.ds(sid * rows_per, rows_per)],
                        out_hbm.at[cid, pl.ds(sid * rows_per, rows_per)])

    return _k


def _sc_scatter(msg, idx, zeros):
    return _sc_scatter_kernel()(msg, idx, zeros)


@functools.lru_cache(maxsize=None)
def _sc_deg_kernel():
    @functools.partial(
        pl.kernel,
        out_type=jax.ShapeDtypeStruct((2, N_ACC, 16), jnp.float32),
        mesh=_sc_mesh(),
        compiler_params=pltpu.CompilerParams(use_tc_tiling_on_sc=False),
        scratch_types=[
            pltpu.VMEM((CH,), jnp.int32),
            pltpu.VMEM((CH, 16), jnp.float32),
            pltpu.VMEM_SHARED((N_ACC, 16), jnp.float32),
        ],
    )
    def _k(idx_hbm, ones_hbm, zero_hbm, out_hbm, idx_v, ones_v, acc):
        cid = lax.axis_index("c")
        sid = lax.axis_index("s")
        wid = sid * 2 + cid
        rows_per = N_ACC // 16
        pltpu.sync_copy(zero_hbm.at[pl.ds(sid * rows_per, rows_per)],
                        acc.at[pl.ds(sid * rows_per, rows_per)])
        pltpu.sync_copy(ones_hbm, ones_v)
        plsc.subcore_barrier()

        def chunk(j, carry):
            base = wid * EPW + j * CH
            pltpu.sync_copy(idx_hbm.at[pl.ds(base, CH)], idx_v)
            pltpu.sync_copy(ones_v, acc.at[idx_v], add=True)
            return carry

        lax.fori_loop(0, NCHUNK, chunk, 0)
        plsc.subcore_barrier()
        pltpu.sync_copy(acc.at[pl.ds(sid * rows_per, rows_per)],
                        out_hbm.at[cid, pl.ds(sid * rows_per, rows_per)])

    return _k


def _sc_deg(idx, ones, zeros):
    return _sc_deg_kernel()(idx, ones, zeros)


def kernel(x, edge_index, edge_attr, batch, params):
    p = params
    f32 = jnp.float32
    pad_e = E_PAD - E
    src_p = jnp.concatenate([edge_index[0].astype(jnp.int32),
                             jnp.zeros((pad_e,), jnp.int32)])
    dst_p = jnp.concatenate([edge_index[1].astype(jnp.int32),
                             jnp.full((pad_e,), N, jnp.int32)])
    ea_p = jnp.concatenate([edge_attr.astype(f32),
                            jnp.zeros((pad_e, 1), f32)], axis=0)

    u1, s1, q1 = _pre1(x, p['pre_W1'].T, p['pre_b1'].reshape(1, 80))
    mu1 = s1 / N
    var1 = q1 / N - mu1 * mu1
    a1 = p['pre_g1'].reshape(1, 80) / jnp.sqrt(var1 + 1e-5)
    c1 = p['pre_beta1'].reshape(1, 80) - mu1 * a1
    ac1 = jnp.concatenate([a1, c1], axis=0)
    u2, s2, q2 = _pre2(u1, ac1, p['pre_W2'].T, p['pre_b2'].reshape(1, P2))
    mu2 = s2 / N
    var2 = q2 / N - mu2 * mu2
    a2 = p['pre_g2'].reshape(1, P2) / jnp.sqrt(var2 + 1e-5)
    c2 = p['pre_beta2'].reshape(1, P2) - mu2 * a2
    out = _prefin(u2, jnp.concatenate([a2, c2], axis=0))

    w01 = p['enn_W1'].T
    es, eq = _e1stats(ea_p, w01)
    emu = es / E
    evar = eq / E - emu * emu
    ea1 = p['enn_g1'].reshape(1, P1) / jnp.sqrt(evar + 1e-5)
    ec1 = p['enn_beta1'].reshape(1, P1) - emu * ea1
    e1_bf, gram, msum = _e1mat(ea_p, w01, jnp.concatenate([ea1, ec1], axis=0))
    a2c, c2c = _e2aff(gram, msum, p['enn_W2'],
                      p['enn_g2'].reshape(P2 * P2, 1),
                      p['enn_beta2'].reshape(P2 * P2, 1))
    w2a = p['enn_W2'].T * a2c.reshape(1, P2 * P2)
    cmat = c2c.reshape(P2, P2)

    degp = _sc_deg(dst_p, jnp.ones((CH, 16), f32), jnp.zeros((N_ACC, 16), f32))
    zeros32 = jnp.zeros((N_ACC, P2), f32)

    crw = p['conv_root']
    cb = p['conv_bias'].reshape(1, P2)
    wiht = p['gru_Wih'].T
    whht = p['gru_Whh'].T
    bih = p['gru_bih'].reshape(1, 3 * P2)
    bhh = p['gru_bhh'].reshape(1, 3 * P2)

    h = out
    for _ in range(NL):
        hs = _sc_gather(h, src_p)
        msg = _msg(e1_bf, hs, w2a, cmat)
        aggp = _sc_scatter(msg, dst_p, zeros32)
        h = _gru(aggp, degp, h, crw, cb, wiht, whht, bih, bhh)

    y = _s2s(h, batch.astype(f32).reshape(1, N), batch.astype(f32).reshape(N, 1),
             p['lstm_Wih'].T, p['lstm_Whh'].T,
             p['lstm_bih'].reshape(1, 4 * P2), p['lstm_bhh'].reshape(1, 4 * P2),
             p['lin1_W'].T, p['lin1_b'].reshape(1, P2),
             p['lin2_W'].T, p['lin2_b'].reshape(1, P2 // 2),
             p['linf_W'].T, p['linf_b'].reshape(1, 1))
    return y.reshape(B)


# speedup vs baseline: 2.6180x; 2.3182x over previous
import functools

import jax
import jax.numpy as jnp
from jax import lax
from jax.experimental import pallas as pl
from jax.experimental.pallas import tpu as pltpu
from jax.experimental.pallas import tpu_sc as plsc

N = 10000
E = 160000
B = 64
NF = 128
P1 = 128
P2 = 32
NL = 3

NW = 32
CH = 128
EPW = 5120
E_PAD = NW * EPW
NCHUNK = EPW // CH
N_ACC = 10240
EB = 2048
NEB = E_PAD // EB
NB_N = 10
NBR = N // NB_N


def _leaky(t):
    return jnp.where(t >= 0, t, 0.01 * t)


def _pre1_body(x_ref, w_ref, b_ref, u_ref, s_ref, q_ref):
    i = pl.program_id(0)
    u = jnp.dot(x_ref[...], w_ref[...], preferred_element_type=jnp.float32)
    u = u + b_ref[...]
    u_ref[...] = u

    @pl.when(i == 0)
    def _():
        s_ref[...] = jnp.zeros_like(s_ref)
        q_ref[...] = jnp.zeros_like(q_ref)

    s_ref[...] += jnp.sum(u, axis=0, keepdims=True)
    q_ref[...] += jnp.sum(u * u, axis=0, keepdims=True)


def _pre1(x, w1t, b1):
    return pl.pallas_call(
        _pre1_body,
        grid=(NB_N,),
        in_specs=[
            pl.BlockSpec((NBR, NF), lambda i: (i, 0)),
            pl.BlockSpec((NF, 80), lambda i: (0, 0)),
            pl.BlockSpec((1, 80), lambda i: (0, 0)),
        ],
        out_specs=[
            pl.BlockSpec((NBR, 80), lambda i: (i, 0)),
            pl.BlockSpec((1, 80), lambda i: (0, 0)),
            pl.BlockSpec((1, 80), lambda i: (0, 0)),
        ],
        out_shape=[
            jax.ShapeDtypeStruct((N, 80), jnp.float32),
            jax.ShapeDtypeStruct((1, 80), jnp.float32),
            jax.ShapeDtypeStruct((1, 80), jnp.float32),
        ],
    )(x, w1t, b1)


def _pre2_body(u_ref, ac_ref, w_ref, b_ref, u2_ref, s_ref, q_ref):
    i = pl.program_id(0)
    h0 = _leaky(u_ref[...] * ac_ref[0:1, :] + ac_ref[1:2, :])
    u2 = jnp.dot(h0, w_ref[...], preferred_element_type=jnp.float32) + b_ref[...]
    u2_ref[...] = u2

    @pl.when(i == 0)
    def _():
        s_ref[...] = jnp.zeros_like(s_ref)
        q_ref[...] = jnp.zeros_like(q_ref)

    s_ref[...] += jnp.sum(u2, axis=0, keepdims=True)
    q_ref[...] += jnp.sum(u2 * u2, axis=0, keepdims=True)


def _pre2(u1, ac, w2t, b2):
    return pl.pallas_call(
        _pre2_body,
        grid=(NB_N,),
        in_specs=[
            pl.BlockSpec((NBR, 80), lambda i: (i, 0)),
            pl.BlockSpec((2, 80), lambda i: (0, 0)),
            pl.BlockSpec((80, P2), lambda i: (0, 0)),
            pl.BlockSpec((1, P2), lambda i: (0, 0)),
        ],
        out_specs=[
            pl.BlockSpec((NBR, P2), lambda i: (i, 0)),
            pl.BlockSpec((1, P2), lambda i: (0, 0)),
            pl.BlockSpec((1, P2), lambda i: (0, 0)),
        ],
        out_shape=[
            jax.ShapeDtypeStruct((N, P2), jnp.float32),
            jax.ShapeDtypeStruct((1, P2), jnp.float32),
            jax.ShapeDtypeStruct((1, P2), jnp.float32),
        ],
    )(u1, ac, w2t, b2)


def _prefin_body(u_ref, ac_ref, o_ref):
    o_ref[...] = _leaky(u_ref[...] * ac_ref[0:1, :] + ac_ref[1:2, :])


def _prefin(u2, ac):
    return pl.pallas_call(
        _prefin_body,
        out_shape=jax.ShapeDtypeStruct((N, P2), jnp.float32),
    )(u2, ac)


def _e1stats_body(r_ref, w01_ref, s_ref, q_ref):
    i = pl.program_id(0)
    r = r_ref[...]
    pre = r * w01_ref[0:1, :] + (1.0 - r) * w01_ref[1:2, :]
    row = i * EB + lax.broadcasted_iota(jnp.int32, (EB, 1), 0)
    pre = pre * (row < E).astype(jnp.float32)

    @pl.when(i == 0)
    def _():
        s_ref[...] = jnp.zeros_like(s_ref)
        q_ref[...] = jnp.zeros_like(q_ref)

    s_ref[...] += jnp.sum(pre, axis=0, keepdims=True)
    q_ref[...] += jnp.sum(pre * pre, axis=0, keepdims=True)


def _e1stats(ea_p, w01):
    return pl.pallas_call(
        _e1stats_body,
        grid=(NEB,),
        in_specs=[
            pl.BlockSpec((EB, 1), lambda i: (i, 0)),
            pl.BlockSpec((2, P1), lambda i: (0, 0)),
        ],
        out_specs=[
            pl.BlockSpec((1, P1), lambda i: (0, 0)),
            pl.BlockSpec((1, P1), lambda i: (0, 0)),
        ],
        out_shape=[
            jax.ShapeDtypeStruct((1, P1), jnp.float32),
            jax.ShapeDtypeStruct((1, P1), jnp.float32),
        ],
    )(ea_p, w01)


def _e1mat_body(r_ref, w01_ref, ac_ref, e1_ref, g_ref, m_ref):
    i = pl.program_id(0)
    r = r_ref[...]
    pre = r * w01_ref[0:1, :] + (1.0 - r) * w01_ref[1:2, :]
    e1 = _leaky(pre * ac_ref[0:1, :] + ac_ref[1:2, :])
    row = i * EB + lax.broadcasted_iota(jnp.int32, (EB, 1), 0)
    e1 = e1 * (row < E).astype(jnp.float32)
    e1_ref[...] = e1

    @pl.when(i == 0)
    def _():
        g_ref[...] = jnp.zeros_like(g_ref)
        m_ref[...] = jnp.zeros_like(m_ref)

    g_ref[...] += lax.dot_general(e1, e1, (((0,), (0,)), ((), ())),
                                  preferred_element_type=jnp.float32)
    m_ref[...] += jnp.sum(e1, axis=0, keepdims=True)


def _e1mat(ea_p, w01, ac):
    return pl.pallas_call(
        _e1mat_body,
        grid=(NEB,),
        in_specs=[
            pl.BlockSpec((EB, 1), lambda i: (i, 0)),
            pl.BlockSpec((2, P1), lambda i: (0, 0)),
            pl.BlockSpec((2, P1), lambda i: (0, 0)),
        ],
        out_specs=[
            pl.BlockSpec((EB, P1), lambda i: (i, 0)),
            pl.BlockSpec((P1, P1), lambda i: (0, 0)),
            pl.BlockSpec((1, P1), lambda i: (0, 0)),
        ],
        out_shape=[
            jax.ShapeDtypeStruct((E_PAD, P1), jnp.float32),
            jax.ShapeDtypeStruct((P1, P1), jnp.float32),
            jax.ShapeDtypeStruct((1, P1), jnp.float32),
        ],
    )(ea_p, w01, ac)


def _e2aff_body(g_ref, m_ref, w2_ref, g2_ref, b2_ref, a_ref, c_ref):
    w2 = w2_ref[...]
    mu = lax.dot_general(w2, m_ref[...], (((1,), (1,)), ((), ())),
                         preferred_element_type=jnp.float32) / E
    t = lax.dot_general(w2, g_ref[...], (((1,), (0,)), ((), ())),
                        preferred_element_type=jnp.float32)
    ssq = jnp.sum(t * w2, axis=1, keepdims=True) / E
    var = ssq - mu * mu
    a = g2_ref[...] / jnp.sqrt(var + 1e-5)
    a_ref[...] = a
    c_ref[...] = b2_ref[...] - mu * a


def _e2aff(g, m, w2, g2c, b2c):
    return pl.pallas_call(
        _e2aff_body,
        out_shape=[
            jax.ShapeDtypeStruct((P2 * P2, 1), jnp.float32),
            jax.ShapeDtypeStruct((P2 * P2, 1), jnp.float32),
        ],
    )(g, m, w2, g2c, b2c)


def _msg_body(e1_ref, hs_ref, w2a_ref, cmat_ref, rexp_ref, ssel_ref, msg_ref):
    e2a = jnp.dot(e1_ref[...], w2a_ref[...],
                  preferred_element_type=jnp.float32)
    hs = hs_ref[...]
    hsw = jnp.dot(hs, rexp_ref[...], preferred_element_type=jnp.float32)
    prod = hsw * e2a
    msg_ref[...] = (
        jnp.dot(prod, ssel_ref[...], preferred_element_type=jnp.float32)
        + jnp.dot(hs, cmat_ref[...], preferred_element_type=jnp.float32))


def _msg(e1_bf, hs, w2a, cmat, rexp, ssel):
    return pl.pallas_call(
        _msg_body,
        grid=(NEB,),
        in_specs=[
            pl.BlockSpec((EB, P1), lambda i: (i, 0)),
            pl.BlockSpec((EB, P2), lambda i: (i, 0)),
            pl.BlockSpec((P1, P2 * P2), lambda i: (0, 0)),
            pl.BlockSpec((P2, P2), lambda i: (0, 0)),
            pl.BlockSpec((P2, P2 * P2), lambda i: (0, 0)),
            pl.BlockSpec((P2 * P2, P2), lambda i: (0, 0)),
        ],
        out_specs=pl.BlockSpec((EB, P2), lambda i: (i, 0)),
        out_shape=jax.ShapeDtypeStruct((E_PAD, P2), jnp.float32),
    )(e1_bf, hs, w2a, cmat, rexp, ssel)


def _gru_body(aggp_ref, degp_ref, h_ref, crw_ref, cb_ref, wih_ref, whh_ref,
              bih_ref, bhh_ref, out_ref):
    agg = aggp_ref[0, 0:N, :] + aggp_ref[1, 0:N, :]
    deg = degp_ref[0, 0:N, 0:1] + degp_ref[1, 0:N, 0:1]
    invd = 1.0 / jnp.maximum(deg, 1.0)
    h = h_ref[...]
    m = _leaky(agg * invd
               + jnp.dot(h, crw_ref[...], preferred_element_type=jnp.float32)
               + cb_ref[...])
    gi = jnp.dot(m, wih_ref[...], preferred_element_type=jnp.float32) + bih_ref[...]
    gh = jnp.dot(h, whh_ref[...], preferred_element_type=jnp.float32) + bhh_ref[...]
    r = jax.nn.sigmoid(gi[:, :P2] + gh[:, :P2])
    z = jax.nn.sigmoid(gi[:, P2:2 * P2] + gh[:, P2:2 * P2])
    n = jnp.tanh(gi[:, 2 * P2:] + r * gh[:, 2 * P2:])
    out_ref[...] = (1.0 - z) * n + z * h


def _gru(aggp, degp, h, crw, cb, wiht, whht, bih, bhh):
    return pl.pallas_call(
        _gru_body,
        out_shape=jax.ShapeDtypeStruct((N, P2), jnp.float32),
    )(aggp, degp, h, crw, cb, wiht, whht, bih, bhh)


def _s2s_body(h_ref, brow_ref, bcol_ref, wih_ref, whh_ref, bih_ref, bhh_ref,
              l1_ref, l1b_ref, l2_ref, l2b_ref, lf_ref, lfb_ref, y_ref):
    h = h_ref[...]
    iota_bn = lax.broadcasted_iota(jnp.int32, (B, N), 0).astype(jnp.float32)
    oh_t = (brow_ref[...] == iota_bn).astype(jnp.float32)
    iota_nb = lax.broadcasted_iota(jnp.int32, (N, B), 1).astype(jnp.float32)
    oh = (bcol_ref[...] == iota_nb).astype(jnp.float32)
    q_star = jnp.zeros((B, 2 * P2), jnp.float32)
    hl = jnp.zeros((B, P2), jnp.float32)
    cl = jnp.zeros((B, P2), jnp.float32)
    for _ in range(3):
        g = (jnp.dot(q_star, wih_ref[...], preferred_element_type=jnp.float32)
             + bih_ref[...]
             + jnp.dot(hl, whh_ref[...], preferred_element_type=jnp.float32)
             + bhh_ref[...])
        i_g = jax.nn.sigmoid(g[:, :P2])
        f_g = jax.nn.sigmoid(g[:, P2:2 * P2])
        g_g = jnp.tanh(g[:, 2 * P2:3 * P2])
        o_g = jax.nn.sigmoid(g[:, 3 * P2:])
        cl = f_g * cl + i_g * g_g
        hl = o_g * jnp.tanh(cl)
        q = hl
        qb = jnp.dot(oh, q, preferred_element_type=jnp.float32)
        e = jnp.sum(h * qb, axis=1, keepdims=True)
        ex = jnp.exp(e - jnp.max(e))
        zc = jnp.concatenate([ex, ex * h], axis=1)
        s = jnp.dot(oh_t, zc, preferred_element_type=jnp.float32)
        r_ = s[:, 1:] / jnp.maximum(s[:, 0:1], 1e-30)
        q_star = jnp.concatenate([q, r_], axis=1)
    y = _leaky(jnp.dot(q_star, l1_ref[...], preferred_element_type=jnp.float32)
               + l1b_ref[...])
    y = _leaky(jnp.dot(y, l2_ref[...], preferred_element_type=jnp.float32)
               + l2b_ref[...])
    y_ref[...] = (jnp.dot(y, lf_ref[...], preferred_element_type=jnp.float32)
                  + lfb_ref[...])


def _s2s(h, brow, bcol, wiht, whht, bih, bhh, l1t, l1b, l2t, l2b, lft, lfb):
    return pl.pallas_call(
        _s2s_body,
        out_shape=jax.ShapeDtypeStruct((B, 1), jnp.float32),
    )(h, brow, bcol, wiht, whht, bih, bhh, l1t, l1b, l2t, l2b, lft, lfb)


def _sc_mesh():
    return plsc.VectorSubcoreMesh(core_axis_name="c", subcore_axis_name="s",
                                  num_cores=2, num_subcores=16)


@functools.lru_cache(maxsize=None)
def _sc_gather_kernel():
    @functools.partial(
        pl.kernel,
        out_type=jax.ShapeDtypeStruct((E_PAD, P2), jnp.float32),
        mesh=_sc_mesh(),
        compiler_params=pltpu.CompilerParams(use_tc_tiling_on_sc=False),
        scratch_types=[
            pltpu.VMEM((CH,), jnp.int32),
            pltpu.VMEM((CH, P2), jnp.float32),
            pltpu.SemaphoreType.DMA,
        ],
    )
    def _k(h_hbm, idx_hbm, out_hbm, idx_v, rows_v, sem):
        wid = lax.axis_index("s") * 2 + lax.axis_index("c")

        def chunk(j, carry):
            base = wid * EPW + j * CH
            pltpu.sync_copy(idx_hbm.at[pl.ds(base, CH)], idx_v)
            pltpu.async_copy(h_hbm.at[idx_v], rows_v, sem).wait()
            pltpu.sync_copy(rows_v, out_hbm.at[pl.ds(base, CH)])
            return carry

        lax.fori_loop(0, NCHUNK, chunk, 0)

    return _k


def _sc_gather(h, idx):
    return _sc_gather_kernel()(h, idx)


@functools.lru_cache(maxsize=None)
def _sc_scatter_kernel():
    @functools.partial(
        pl.kernel,
        out_type=jax.ShapeDtypeStruct((2, N_ACC, P2), jnp.float32),
        mesh=_sc_mesh(),
        compiler_params=pltpu.CompilerParams(use_tc_tiling_on_sc=False),
        scratch_types=[
            pltpu.VMEM((CH,), jnp.int32),
            pltpu.VMEM((CH, P2), jnp.float32),
            pltpu.VMEM_SHARED((N_ACC, P2), jnp.float32),
        ],
    )
    def _k(msg_hbm, idx_hbm, zero_hbm, out_hbm, idx_v, rows_v, acc):
        cid = lax.axis_index("c")
        sid = lax.axis_index("s")
        wid = sid * 2 + cid
        rows_per = N_ACC // 16
        pltpu.sync_copy(zero_hbm.at[pl.ds(sid * rows_per, rows_per)],
                        acc.at[pl.ds(sid * rows_per, rows_per)])
        plsc.subcore_barrier()

        def chunk(j, carry):
            base = wid * EPW + j * CH
            pltpu.sync_copy(idx_hbm.at[pl.ds(base, CH)], idx_v)
            pltpu.sync_copy(msg_hbm.at[pl.ds(base, CH)], rows_v)
            pltpu.sync_copy(rows_v, acc.at[idx_v], add=True)
            return carry

        lax.fori_loop(0, NCHUNK, chunk, 0)
        plsc.subcore_barrier()
        pltpu.sync_copy(acc.at[pl.ds(sid * rows_per, rows_per)],
                        out_hbm.at[cid, pl.ds(sid * rows_per, rows_per)])

    return _k


def _sc_scatter(msg, idx, zeros):
    return _sc_scatter_kernel()(msg, idx, zeros)


@functools.lru_cache(maxsize=None)
def _sc_deg_kernel():
    @functools.partial(
        pl.kernel,
        out_type=jax.ShapeDtypeStruct((2, N_ACC, 16), jnp.float32),
        mesh=_sc_mesh(),
        compiler_params=pltpu.CompilerParams(use_tc_tiling_on_sc=False),
        scratch_types=[
            pltpu.VMEM((CH,), jnp.int32),
            pltpu.VMEM((CH, 16), jnp.float32),
            pltpu.VMEM_SHARED((N_ACC, 16), jnp.float32),
        ],
    )
    def _k(idx_hbm, ones_hbm, zero_hbm, out_hbm, idx_v, ones_v, acc):
        cid = lax.axis_index("c")
        sid = lax.axis_index("s")
        wid = sid * 2 + cid
        rows_per = N_ACC // 16
        pltpu.sync_copy(zero_hbm.at[pl.ds(sid * rows_per, rows_per)],
                        acc.at[pl.ds(sid * rows_per, rows_per)])
        pltpu.sync_copy(ones_hbm, ones_v)
        plsc.subcore_barrier()

        def chunk(j, carry):
            base = wid * EPW + j * CH
            pltpu.sync_copy(idx_hbm.at[pl.ds(base, CH)], idx_v)
            pltpu.sync_copy(ones_v, acc.at[idx_v], add=True)
            return carry

        lax.fori_loop(0, NCHUNK, chunk, 0)
        plsc.subcore_barrier()
        pltpu.sync_copy(acc.at[pl.ds(sid * rows_per, rows_per)],
                        out_hbm.at[cid, pl.ds(sid * rows_per, rows_per)])

    return _k


def _sc_deg(idx, ones, zeros):
    return _sc_deg_kernel()(idx, ones, zeros)


def kernel(x, edge_index, edge_attr, batch, params):
    p = params
    f32 = jnp.float32
    pad_e = E_PAD - E
    src_p = jnp.concatenate([edge_index[0].astype(jnp.int32),
                             jnp.zeros((pad_e,), jnp.int32)])
    dst_p = jnp.concatenate([edge_index[1].astype(jnp.int32),
                             jnp.full((pad_e,), N, jnp.int32)])
    ea_p = jnp.concatenate([edge_attr.astype(f32),
                            jnp.zeros((pad_e, 1), f32)], axis=0)

    u1, s1, q1 = _pre1(x, p['pre_W1'].T, p['pre_b1'].reshape(1, 80))
    mu1 = s1 / N
    var1 = q1 / N - mu1 * mu1
    a1 = p['pre_g1'].reshape(1, 80) / jnp.sqrt(var1 + 1e-5)
    c1 = p['pre_beta1'].reshape(1, 80) - mu1 * a1
    ac1 = jnp.concatenate([a1, c1], axis=0)
    u2, s2, q2 = _pre2(u1, ac1, p['pre_W2'].T, p['pre_b2'].reshape(1, P2))
    mu2 = s2 / N
    var2 = q2 / N - mu2 * mu2
    a2 = p['pre_g2'].reshape(1, P2) / jnp.sqrt(var2 + 1e-5)
    c2 = p['pre_beta2'].reshape(1, P2) - mu2 * a2
    out = _prefin(u2, jnp.concatenate([a2, c2], axis=0))

    w01 = p['enn_W1'].T
    es, eq = _e1stats(ea_p, w01)
    emu = es / E
    evar = eq / E - emu * emu
    ea1 = p['enn_g1'].reshape(1, P1) / jnp.sqrt(evar + 1e-5)
    ec1 = p['enn_beta1'].reshape(1, P1) - emu * ea1
    e1_bf, gram, msum = _e1mat(ea_p, w01, jnp.concatenate([ea1, ec1], axis=0))
    a2c, c2c = _e2aff(gram, msum, p['enn_W2'],
                      p['enn_g2'].reshape(P2 * P2, 1),
                      p['enn_beta2'].reshape(P2 * P2, 1))
    w2a = p['enn_W2'].T * a2c.reshape(1, P2 * P2)
    cmat = c2c.reshape(P2, P2)
    lanes = jnp.arange(P2 * P2)
    rexp = (lanes[None, :] // P2 == jnp.arange(P2)[:, None]).astype(f32)
    ssel = (lanes[:, None] % P2 == jnp.arange(P2)[None, :]).astype(f32)

    degp = _sc_deg(dst_p, jnp.ones((CH, 16), f32), jnp.zeros((N_ACC, 16), f32))
    zeros32 = jnp.zeros((N_ACC, P2), f32)

    crw = p['conv_root']
    cb = p['conv_bias'].reshape(1, P2)
    wiht = p['gru_Wih'].T
    whht = p['gru_Whh'].T
    bih = p['gru_bih'].reshape(1, 3 * P2)
    bhh = p['gru_bhh'].reshape(1, 3 * P2)

    h = out
    for _ in range(NL):
        hs = _sc_gather(h, src_p)
        msg = _msg(e1_bf, hs, w2a, cmat, rexp, ssel)
        aggp = _sc_scatter(msg, dst_p, zeros32)
        h = _gru(aggp, degp, h, crw, cb, wiht, whht, bih, bhh)

    y = _s2s(h, batch.astype(f32).reshape(1, N), batch.astype(f32).reshape(N, 1),
             p['lstm_Wih'].T, p['lstm_Whh'].T,
             p['lstm_bih'].reshape(1, 4 * P2), p['lstm_bhh'].reshape(1, 4 * P2),
             p['lin1_W'].T, p['lin1_b'].reshape(1, P2),
             p['lin2_W'].T, p['lin2_b'].reshape(1, P2 // 2),
             p['linf_W'].T, p['linf_b'].reshape(1, 1))
    return y.reshape(B)
